# baseline (speedup 1.0000x reference)
import jax
import jax.numpy as jnp
from jax.experimental import pallas as pl
from jax.experimental.pallas import tpu as pltpu

B, T, F, D, H = 16, 20, 128, 768, 1024
BT = B * T
BI = 32
BS = 16
NI = F // BI
NBO = 6
NBD = 12

_OFF = [(i, j) for i in range(NI) for j in range(NI) if j > i]
_DIA = []
for d in range(NI):
    base = d * BI
    _DIA += [(base, base, True),
             (base, base + BS, False),
             (base + BS, base + BS, True)]


def _fused_kernel(mask_ref, e_hbm, w_hbm, b_ref, out_ref,
                  acc_ref, ebo_ref, ebd_ref, wv_ref, mt_ref,
                  osems, dsems, wsem):

    def ostart(k):
        i0, j0 = _OFF[k]
        pltpu.make_async_copy(
            e_hbm.at[pl.ds(i0 * BI, BI), pl.ds(j0 * BI, BI), :],
            ebo_ref.at[k % NBO], osems.at[k % NBO]).start()

    def owait(k):
        pltpu.make_async_copy(
            e_hbm.at[pl.ds(0, BI), pl.ds(0, BI), :],
            ebo_ref.at[k % NBO], osems.at[k % NBO]).wait()

    def dstart(k):
        r0, c0, _ = _DIA[k]
        pltpu.make_async_copy(
            e_hbm.at[pl.ds(r0, BS), pl.ds(c0, BS), :],
            ebd_ref.at[k % NBD], dsems.at[k % NBD]).start()

    def dwait(k):
        pltpu.make_async_copy(
            e_hbm.at[pl.ds(0, BS), pl.ds(0, BS), :],
            ebd_ref.at[k % NBD], dsems.at[k % NBD]).wait()

    for k in range(len(_OFF)):
        ostart(k)
    for k in range(len(_DIA)):
        dstart(k)
    pltpu.make_async_copy(w_hbm, wv_ref, wsem).start()

    mf = mask_ref[...].astype(jnp.float32)
    mt_ref[...] = mf.T
    ti = jax.lax.broadcasted_iota(jnp.int32, (BS, BS, 1), 0)
    tj = jax.lax.broadcasted_iota(jnp.int32, (BS, BS, 1), 1)
    triu = (ti < tj).astype(jnp.float32)

    for k in range(len(_OFF)):
        owait(k)
        i0, j0 = _OFF[k]
        mi = mt_ref[i0 * BI:(i0 + 1) * BI, :]
        mj = mt_ref[j0 * BI:(j0 + 1) * BI, :]
        pt = mi[:, None, :] * mj[None, :, :]
        pt2 = pt.reshape(BI * BI, BT).astype(jnp.bfloat16)
        e2 = ebo_ref[k % NBO].reshape(BI * BI, D).astype(jnp.bfloat16)
        d = jax.lax.dot_general(
            pt2, e2, (((0,), (0,)), ((), ())),
            preferred_element_type=jnp.float32)
        if k == 0:
            acc_ref[...] = d
        else:
            acc_ref[...] += d

    for k in range(len(_DIA)):
        dwait(k)
        r0, c0, needs_tri = _DIA[k]
        mi = mt_ref[r0:r0 + BS, :]
        mj = mt_ref[c0:c0 + BS, :]
        pt = mi[:, None, :] * mj[None, :, :]
        if needs_tri:
            pt = pt * triu
        pt2 = pt.reshape(BS * BS, BT).astype(jnp.bfloat16)
        e2 = ebd_ref[k % NBD].reshape(BS * BS, D).astype(jnp.bfloat16)
        acc_ref[...] += jax.lax.dot_general(
            pt2, e2, (((0,), (0,)), ((), ())),
            preferred_element_type=jnp.float32)

    s = jnp.sum(mf, axis=1, keepdims=True)
    cnt = (s * s - s) * 0.5
    inv = jnp.where(cnt > 0, 1.0 / jnp.maximum(cnt, 1.0), 0.0)
    pltpu.make_async_copy(w_hbm, wv_ref, wsem).wait()
    raw = jax.lax.dot_general(
        acc_ref[...], wv_ref[...], (((1,), (1,)), ((), ())),
        preferred_element_type=jnp.float32)
    out_ref[...] = raw * inv + b_ref[...]


def kernel(surviving_mask, precomputed_embeddings, variable_indices, W, b):
    mask2d = surviving_mask.reshape(BT, F)

    out = pl.pallas_call(
        _fused_kernel,
        in_specs=[
            pl.BlockSpec((BT, F), lambda: (0, 0)),
            pl.BlockSpec(memory_space=pltpu.MemorySpace.HBM),
            pl.BlockSpec(memory_space=pltpu.MemorySpace.HBM),
            pl.BlockSpec((1, H), lambda: (0, 0)),
        ],
        out_specs=pl.BlockSpec((BT, H), lambda: (0, 0)),
        scratch_shapes=[
            pltpu.VMEM((BT, D), jnp.float32),
            pltpu.VMEM((NBO, BI, BI, D), jnp.float32),
            pltpu.VMEM((NBD, BS, BS, D), jnp.float32),
            pltpu.VMEM((H, D), jnp.float32),
            pltpu.VMEM((F, BT), jnp.float32),
            pltpu.SemaphoreType.DMA((NBO,)),
            pltpu.SemaphoreType.DMA((NBD,)),
            pltpu.SemaphoreType.DMA,
        ],
        out_shape=jax.ShapeDtypeStruct((BT, H), jnp.float32),
    )(mask2d, precomputed_embeddings, W, b.reshape(1, H))

    return out.reshape(B, T, H)


# speedup vs baseline: 1.1278x; 1.1278x over previous
import jax
import jax.numpy as jnp
from jax.experimental import pallas as pl
from jax.experimental.pallas import tpu as pltpu

B, T, F, D, H = 16, 20, 128, 768, 1024
BT = B * T
BI = 32
BS = 16
NI = F // BI

_OFF = [(r * BI, (r + 1) * BI, F - (r + 1) * BI) for r in range(NI - 1)]


def _fused_kernel(mask_ref, e_hbm, w_hbm, b_ref, out_ref,
                  acc_ref, eo0_ref, eo1_ref, eo2_ref, eda_ref, edb_ref,
                  wv_ref, mt_ref, osems, asems, bsems, wsem):
    obufs = [eo0_ref, eo1_ref, eo2_ref]

    def ostart(k, wait=False):
        r0, c0, nc = _OFF[k]
        cp = pltpu.make_async_copy(
            e_hbm.at[pl.ds(r0, BI), pl.ds(c0, nc), :],
            obufs[k], osems.at[k])
        cp.wait() if wait else cp.start()

    def dastart(d, wait=False):
        base = d * BI
        cp = pltpu.make_async_copy(
            e_hbm.at[pl.ds(base, BS), pl.ds(base, 2 * BS), :],
            eda_ref.at[d], asems.at[d])
        cp.wait() if wait else cp.start()

    def dbstart(d, wait=False):
        base = d * BI + BS
        cp = pltpu.make_async_copy(
            e_hbm.at[pl.ds(base, BS), pl.ds(base, BS), :],
            edb_ref.at[d], bsems.at[d])
        cp.wait() if wait else cp.start()

    for k in range(len(_OFF)):
        ostart(k)
    pltpu.make_async_copy(w_hbm, wv_ref, wsem).start()

    mf = mask_ref[...].astype(jnp.float32)
    mt_ref[...] = mf.T

    for k in range(len(_OFF)):
        ostart(k, wait=True)
        dastart(k)
        dbstart(k)
        r0, c0, nc = _OFF[k]
        mi = mt_ref[r0:r0 + BI, :]
        mj = mt_ref[c0:c0 + nc, :]
        pt = mi[:, None, :] * mj[None, :, :]
        pt2 = pt.reshape(BI * nc, BT).astype(jnp.bfloat16)
        e2 = obufs[k][...].reshape(BI * nc, D).astype(jnp.bfloat16)
        d = jax.lax.dot_general(
            pt2, e2, (((0,), (0,)), ((), ())),
            preferred_element_type=jnp.float32)
        if k == 0:
            acc_ref[...] = d
        else:
            acc_ref[...] += d

    dastart(3)
    dbstart(3)

    for d in range(NI):
        base = d * BI
        dastart(d, wait=True)
        mi = mt_ref[base:base + BS, :]
        mj = mt_ref[base:base + 2 * BS, :]
        ti = jax.lax.broadcasted_iota(jnp.int32, (BS, 2 * BS, 1), 0)
        tj = jax.lax.broadcasted_iota(jnp.int32, (BS, 2 * BS, 1), 1)
        pt = mi[:, None, :] * mj[None, :, :] * (ti < tj).astype(jnp.float32)
        pt2 = pt.reshape(BS * 2 * BS, BT).astype(jnp.bfloat16)
        e2 = eda_ref[d].reshape(BS * 2 * BS, D).astype(jnp.bfloat16)
        acc_ref[...] += jax.lax.dot_general(
            pt2, e2, (((0,), (0,)), ((), ())),
            preferred_element_type=jnp.float32)
        dbstart(d, wait=True)
        mi = mt_ref[base + BS:base + 2 * BS, :]
        mj = mt_ref[base + BS:base + 2 * BS, :]
        ti = jax.lax.broadcasted_iota(jnp.int32, (BS, BS, 1), 0)
        tj = jax.lax.broadcasted_iota(jnp.int32, (BS, BS, 1), 1)
        pt = mi[:, None, :] * mj[None, :, :] * (ti < tj).astype(jnp.float32)
        pt2 = pt.reshape(BS * BS, BT).astype(jnp.bfloat16)
        e2 = edb_ref[d].reshape(BS * BS, D).astype(jnp.bfloat16)
        acc_ref[...] += jax.lax.dot_general(
            pt2, e2, (((0,), (0,)), ((), ())),
            preferred_element_type=jnp.float32)

    s = jnp.sum(mf, axis=1, keepdims=True)
    cnt = (s * s - s) * 0.5
    inv = jnp.where(cnt > 0, 1.0 / jnp.maximum(cnt, 1.0), 0.0)
    pltpu.make_async_copy(w_hbm, wv_ref, wsem).wait()
    raw = jax.lax.dot_general(
        acc_ref[...], wv_ref[...], (((1,), (1,)), ((), ())),
        preferred_element_type=jnp.float32)
    out_ref[...] = raw * inv + b_ref[...]


def kernel(surviving_mask, precomputed_embeddings, variable_indices, W, b):
    mask2d = surviving_mask.reshape(BT, F)

    out = pl.pallas_call(
        _fused_kernel,
        in_specs=[
            pl.BlockSpec((BT, F), lambda: (0, 0)),
            pl.BlockSpec(memory_space=pltpu.MemorySpace.HBM),
            pl.BlockSpec(memory_space=pltpu.MemorySpace.HBM),
            pl.BlockSpec((1, H), lambda: (0, 0)),
        ],
        out_specs=pl.BlockSpec((BT, H), lambda: (0, 0)),
        scratch_shapes=[
            pltpu.VMEM((BT, D), jnp.float32),
            pltpu.VMEM((BI, _OFF[0][2], D), jnp.float32),
            pltpu.VMEM((BI, _OFF[1][2], D), jnp.float32),
            pltpu.VMEM((BI, _OFF[2][2], D), jnp.float32),
            pltpu.VMEM((NI, BS, 2 * BS, D), jnp.float32),
            pltpu.VMEM((NI, BS, BS, D), jnp.float32),
            pltpu.VMEM((H, D), jnp.float32),
            pltpu.VMEM((F, BT), jnp.float32),
            pltpu.SemaphoreType.DMA((len(_OFF),)),
            pltpu.SemaphoreType.DMA((NI,)),
            pltpu.SemaphoreType.DMA((NI,)),
            pltpu.SemaphoreType.DMA,
        ],
        out_shape=jax.ShapeDtypeStruct((BT, H), jnp.float32),
    )(mask2d, precomputed_embeddings, W, b.reshape(1, H))

    return out.reshape(B, T, H)


# speedup vs baseline: 1.1353x; 1.0066x over previous
import jax
import jax.numpy as jnp
from jax.experimental import pallas as pl
from jax.experimental.pallas import tpu as pltpu

B, T, F, D, H = 16, 20, 128, 768, 1024
BT = B * T
BI = 32
BS = 16
NI = F // BI

_OFF = [(r * BI, (r + 1) * BI, F - (r + 1) * BI) for r in range(NI - 1)]


def _fused_kernel(mask_ref, e_hbm, w_hbm, b_ref, out_ref,
                  acc_ref, eo0_ref, eo1_ref, eo2_ref, eda_ref, edb_ref,
                  wv_ref, mt_ref, osems, asems, bsems, wsem):
    obufs = [eo0_ref, eo1_ref, eo2_ref]

    def ostart(k, wait=False):
        r0, c0, nc = _OFF[k]
        cp = pltpu.make_async_copy(
            e_hbm.at[pl.ds(r0, BI), pl.ds(c0, nc), :],
            obufs[k], osems.at[k])
        cp.wait() if wait else cp.start()

    def dastart(d, wait=False):
        base = d * BI
        cp = pltpu.make_async_copy(
            e_hbm.at[pl.ds(base, BS), pl.ds(base, 2 * BS), :],
            eda_ref.at[d], asems.at[d])
        cp.wait() if wait else cp.start()

    def dbstart(d, wait=False):
        base = d * BI + BS
        cp = pltpu.make_async_copy(
            e_hbm.at[pl.ds(base, BS), pl.ds(base, BS), :],
            edb_ref.at[d], bsems.at[d])
        cp.wait() if wait else cp.start()

    for k in range(len(_OFF)):
        ostart(k)

    mf = mask_ref[...].astype(jnp.float32)
    mt_ref[...] = mf.T

    for k in range(len(_OFF)):
        ostart(k, wait=True)
        dastart(k)
        dbstart(k)
        r0, c0, nc = _OFF[k]
        mi = mt_ref[r0:r0 + BI, :]
        mj = mt_ref[c0:c0 + nc, :]
        pt = mi[:, None, :] * mj[None, :, :]
        pt2 = pt.reshape(BI * nc, BT).astype(jnp.bfloat16)
        e2 = obufs[k][...].reshape(BI * nc, D).astype(jnp.bfloat16)
        d = jax.lax.dot_general(
            pt2, e2, (((0,), (0,)), ((), ())),
            preferred_element_type=jnp.float32)
        if k == 0:
            acc_ref[...] = d
        else:
            acc_ref[...] += d

    dastart(3)
    dbstart(3)
    pltpu.make_async_copy(w_hbm, wv_ref, wsem).start()

    for d in range(NI):
        base = d * BI
        dastart(d, wait=True)
        mi = mt_ref[base:base + BS, :]
        mj = mt_ref[base:base + 2 * BS, :]
        ti = jax.lax.broadcasted_iota(jnp.int32, (BS, 2 * BS, 1), 0)
        tj = jax.lax.broadcasted_iota(jnp.int32, (BS, 2 * BS, 1), 1)
        pt = mi[:, None, :] * mj[None, :, :] * (ti < tj).astype(jnp.float32)
        pt2 = pt.reshape(BS * 2 * BS, BT).astype(jnp.bfloat16)
        e2 = eda_ref[d].reshape(BS * 2 * BS, D).astype(jnp.bfloat16)
        acc_ref[...] += jax.lax.dot_general(
            pt2, e2, (((0,), (0,)), ((), ())),
            preferred_element_type=jnp.float32)
        dbstart(d, wait=True)
        mi = mt_ref[base + BS:base + 2 * BS, :]
        mj = mt_ref[base + BS:base + 2 * BS, :]
        ti = jax.lax.broadcasted_iota(jnp.int32, (BS, BS, 1), 0)
        tj = jax.lax.broadcasted_iota(jnp.int32, (BS, BS, 1), 1)
        pt = mi[:, None, :] * mj[None, :, :] * (ti < tj).astype(jnp.float32)
        pt2 = pt.reshape(BS * BS, BT).astype(jnp.bfloat16)
        e2 = edb_ref[d].reshape(BS * BS, D).astype(jnp.bfloat16)
        acc_ref[...] += jax.lax.dot_general(
            pt2, e2, (((0,), (0,)), ((), ())),
            preferred_element_type=jnp.float32)

    s = jnp.sum(mf, axis=1, keepdims=True)
    cnt = (s * s - s) * 0.5
    inv = jnp.where(cnt > 0, 1.0 / jnp.maximum(cnt, 1.0), 0.0)
    pltpu.make_async_copy(w_hbm, wv_ref, wsem).wait()
    raw = jax.lax.dot_general(
        acc_ref[...], wv_ref[...], (((1,), (1,)), ((), ())),
        preferred_element_type=jnp.float32)
    out_ref[...] = raw * inv + b_ref[...]


def kernel(surviving_mask, precomputed_embeddings, variable_indices, W, b):
    mask2d = surviving_mask.reshape(BT, F)

    out = pl.pallas_call(
        _fused_kernel,
        in_specs=[
            pl.BlockSpec((BT, F), lambda: (0, 0)),
            pl.BlockSpec(memory_space=pltpu.MemorySpace.HBM),
            pl.BlockSpec(memory_space=pltpu.MemorySpace.HBM),
            pl.BlockSpec((1, H), lambda: (0, 0)),
        ],
        out_specs=pl.BlockSpec((BT, H), lambda: (0, 0)),
        scratch_shapes=[
            pltpu.VMEM((BT, D), jnp.float32),
            pltpu.VMEM((BI, _OFF[0][2], D), jnp.float32),
            pltpu.VMEM((BI, _OFF[1][2], D), jnp.float32),
            pltpu.VMEM((BI, _OFF[2][2], D), jnp.float32),
            pltpu.VMEM((NI, BS, 2 * BS, D), jnp.float32),
            pltpu.VMEM((NI, BS, BS, D), jnp.float32),
            pltpu.VMEM((H, D), jnp.float32),
            pltpu.VMEM((F, BT), jnp.float32),
            pltpu.SemaphoreType.DMA((len(_OFF),)),
            pltpu.SemaphoreType.DMA((NI,)),
            pltpu.SemaphoreType.DMA((NI,)),
            pltpu.SemaphoreType.DMA,
        ],
        out_shape=jax.ShapeDtypeStruct((BT, H), jnp.float32),
    )(mask2d, precomputed_embeddings, W, b.reshape(1, H))

    return out.reshape(B, T, H)
